# baseline (speedup 1.0000x reference)
import functools

import jax
import jax.numpy as jnp
from jax import lax
from jax.experimental import pallas as pl
from jax.experimental.pallas import tpu as pltpu
from jax.experimental.pallas import tpu_sc as plsc

BATCH = 16384
DIM = 16
NUM_WORKERS = 32
B_PER_W = BATCH // NUM_WORKERS
GATHER_CHUNK = 128
N_CHUNKS = B_PER_W // GATHER_CHUNK


def _body(users_hbm, items_hbm, eu_hbm, ei_hbm, out_hbm,
          uidx, iidx, urows, irows, outv, sem):
    wid = lax.axis_index("s") * 2 + lax.axis_index("c")
    base = wid * B_PER_W

    pltpu.sync_copy(users_hbm.at[pl.ds(base, B_PER_W)], uidx)
    pltpu.sync_copy(items_hbm.at[pl.ds(base, B_PER_W)], iidx)

    copies = []
    for ch in range(N_CHUNKS):
        sl = pl.ds(ch * GATHER_CHUNK, GATHER_CHUNK)
        copies.append(pltpu.async_copy(eu_hbm.at[uidx.at[sl]], urows.at[sl], sem))
        copies.append(pltpu.async_copy(ei_hbm.at[iidx.at[sl]], irows.at[sl], sem))
    for c in copies:
        c.wait()

    iota16 = lax.iota(jnp.int32, DIM)

    def chunk(c, carry):
        ridx = c * DIM + iota16
        acc = jnp.zeros((DIM,), jnp.float32)
        for l in range(DIM):
            lidx = jnp.full((DIM,), l, jnp.int32)
            ul = plsc.load_gather(urows, [ridx, lidx])
            il = plsc.load_gather(irows, [ridx, lidx])
            acc = acc + ul * il
        sig = 1.0 / (1.0 + jnp.exp(-acc))
        outv[pl.ds(pl.multiple_of(c * DIM, DIM), DIM)] = sig
        return carry

    lax.fori_loop(0, B_PER_W // DIM, chunk, 0)

    pltpu.sync_copy(outv, out_hbm.at[pl.ds(base, B_PER_W)])


@jax.jit
def _bpr(users, items, embedding_user, embedding_item):
    mesh = plsc.VectorSubcoreMesh(core_axis_name="c", subcore_axis_name="s")
    run = functools.partial(
        pl.kernel,
        out_type=jax.ShapeDtypeStruct((BATCH,), jnp.float32),
        mesh=mesh,
        compiler_params=pltpu.CompilerParams(
            use_tc_tiling_on_sc=False, needs_layout_passes=False),
        scratch_types=[
            pltpu.VMEM((B_PER_W,), jnp.int32),
            pltpu.VMEM((B_PER_W,), jnp.int32),
            pltpu.VMEM((B_PER_W, DIM), jnp.float32),
            pltpu.VMEM((B_PER_W, DIM), jnp.float32),
            pltpu.VMEM((B_PER_W,), jnp.float32),
            pltpu.SemaphoreType.DMA,
        ],
    )(_body)
    return run(users, items, embedding_user, embedding_item)


def kernel(users, items, embedding_user, embedding_item):
    return _bpr(users.astype(jnp.int32), items.astype(jnp.int32),
                embedding_user, embedding_item)
